# initial kernel scaffold (unmeasured)
import jax
import jax.numpy as jnp
from jax import lax
from jax.experimental import pallas as pl
from jax.experimental.pallas import tpu as pltpu

N_DEV = 32
HQ_LOC = 8
DH = 128
SQ = 1024
SKV = 1024
DM = 1024
BLK = 64
SCALE = 0.08838834764831843
CH = SQ // N_DEV
NEG = -1e9


def _attn_body(x_ref, wq_ref, k_ref, v_ref, wo_ref, out_ref):
    h = pl.program_id(0)

    @pl.when(h == 0)
    def _():
        out_ref[...] = jnp.zeros_like(out_ref)

    q = jnp.dot(x_ref[...], wq_ref[...], preferred_element_type=jnp.float32)
    k = k_ref[:, 0, :]
    v = v_ref[:, 0, :]
    s = lax.dot_general(
        q, k, (((1,), (1,)), ((), ())), preferred_element_type=jnp.float32
    ) * SCALE
    qb = lax.broadcasted_iota(jnp.int32, (SQ, SKV), 0) // BLK
    kb = lax.broadcasted_iota(jnp.int32, (SQ, SKV), 1) // BLK
    mask = (qb == kb) | (kb == 0) | (((qb + kb) % 3) == 0)
    s = jnp.where(mask, s, NEG)
    m = jnp.max(s, axis=-1, keepdims=True)
    w = jnp.exp(s - m)
    p = w / jnp.sum(w, axis=-1, keepdims=True)
    ctx = jnp.dot(p, v, preferred_element_type=jnp.float32)
    out_ref[...] += jnp.dot(ctx, wo_ref[...], preferred_element_type=jnp.float32)


def _ar_body(p_ref, out_ref, comm_ref, send_sems, recv_sems, credit_sem):
    i = lax.axis_index("i")
    left = lax.rem(i + N_DEV - 1, N_DEV)
    right = lax.rem(i + 1, N_DEV)

    barrier = pltpu.get_barrier_semaphore()
    for nbr in (left, right):
        pl.semaphore_signal(
            barrier, inc=1, device_id=(nbr,), device_id_type=pl.DeviceIdType.MESH
        )
    pl.semaphore_wait(barrier, 2)

    pl.semaphore_signal(
        credit_sem, inc=2, device_id=(left,), device_id_type=pl.DeviceIdType.MESH
    )

    out_ref[...] = p_ref[...]

    n_steps = 2 * (N_DEV - 1)
    for s in range(n_steps):
        slot = s % 2
        if s < N_DEV - 1:
            send_c = lax.rem(i - s + 2 * N_DEV, N_DEV)
            recv_c = lax.rem(i - s - 1 + 2 * N_DEV, N_DEV)
        else:
            t = s - (N_DEV - 1)
            send_c = lax.rem(i + 1 - t + 2 * N_DEV, N_DEV)
            recv_c = lax.rem(i - t + 2 * N_DEV, N_DEV)
        pl.semaphore_wait(credit_sem, 1)
        rdma = pltpu.make_async_remote_copy(
            src_ref=out_ref.at[send_c],
            dst_ref=comm_ref.at[slot],
            send_sem=send_sems.at[slot],
            recv_sem=recv_sems.at[slot],
            device_id=(right,),
            device_id_type=pl.DeviceIdType.MESH,
        )
        rdma.start()
        rdma.wait()
        if s < N_DEV - 1:
            out_ref[recv_c] = out_ref[recv_c] + comm_ref[slot]
        else:
            out_ref[recv_c] = comm_ref[slot]
        if s < n_steps - 2:
            pl.semaphore_signal(
                credit_sem,
                inc=1,
                device_id=(left,),
                device_id_type=pl.DeviceIdType.MESH,
            )


def kernel(x, Wq, K_ext, V_ext, Wo):
    i = lax.axis_index("i")
    wq_sl = lax.dynamic_slice(Wq, (0, i * DM), (DM, DM))
    wo_sl = lax.dynamic_slice(Wo, (i * DM, 0), (DM, DM))
    xs = x[0]
    k = K_ext[0]
    v = V_ext[0]

    partial = pl.pallas_call(
        _attn_body,
        grid=(HQ_LOC,),
        in_specs=[
            pl.BlockSpec((SQ, DM), lambda h: (0, 0)),
            pl.BlockSpec((DM, DH), lambda h: (0, h)),
            pl.BlockSpec((SKV, 1, DH), lambda h: (0, h, 0)),
            pl.BlockSpec((SKV, 1, DH), lambda h: (0, h, 0)),
            pl.BlockSpec((DH, DM), lambda h: (h, 0)),
        ],
        out_specs=pl.BlockSpec((SQ, DM), lambda h: (0, 0)),
        out_shape=jax.ShapeDtypeStruct((SQ, DM), jnp.float32),
        compiler_params=pltpu.CompilerParams(
            dimension_semantics=("arbitrary",)
        ),
    )(xs, wq_sl, k, v, wo_sl)

    ar_in = partial.reshape(N_DEV, CH, DM)
    out = pl.pallas_call(
        _ar_body,
        out_shape=jax.ShapeDtypeStruct((N_DEV, CH, DM), jnp.float32),
        in_specs=[pl.BlockSpec(memory_space=pltpu.VMEM)],
        out_specs=pl.BlockSpec(memory_space=pltpu.VMEM),
        scratch_shapes=[
            pltpu.VMEM((2, CH, DM), jnp.float32),
            pltpu.SemaphoreType.DMA((2,)),
            pltpu.SemaphoreType.DMA((2,)),
            pltpu.SemaphoreType.REGULAR,
        ],
        compiler_params=pltpu.CompilerParams(collective_id=0),
    )(ar_in)
    return out.reshape(1, SQ, DM)


# baseline (device time: 328423 ns/iter reference)
import jax
import jax.numpy as jnp
from jax import lax
from jax.experimental import pallas as pl
from jax.experimental.pallas import tpu as pltpu

N_DEV = 32
HQ_LOC = 8
DH = 128
SQ = 1024
SKV = 1024
DM = 1024
BLK = 64
SCALE = 0.08838834764831843
CH = SQ // N_DEV
NEG = -1e9


def _attn_body(x_ref, wq_ref, k_ref, v_ref, wo_ref, out_ref):
    h = pl.program_id(0)

    @pl.when(h == 0)
    def _():
        out_ref[...] = jnp.zeros_like(out_ref)

    q = jnp.dot(x_ref[...], wq_ref[...], preferred_element_type=jnp.float32)
    k = k_ref[0]
    v = v_ref[0]
    s = lax.dot_general(
        q, k, (((1,), (1,)), ((), ())), preferred_element_type=jnp.float32
    ) * SCALE
    qb = lax.broadcasted_iota(jnp.int32, (SQ, SKV), 0) // BLK
    kb = lax.broadcasted_iota(jnp.int32, (SQ, SKV), 1) // BLK
    mask = (qb == kb) | (kb == 0) | (((qb + kb) % 3) == 0)
    s = jnp.where(mask, s, NEG)
    m = jnp.max(s, axis=-1, keepdims=True)
    w = jnp.exp(s - m)
    p = w / jnp.sum(w, axis=-1, keepdims=True)
    ctx = jnp.dot(p, v, preferred_element_type=jnp.float32)
    out_ref[...] += jnp.dot(ctx, wo_ref[...], preferred_element_type=jnp.float32)


def _ar_body(p_ref, out_ref, comm_ref, send_sems, recv_sems, credit_sem):
    i = lax.axis_index("i")
    left = lax.rem(i + N_DEV - 1, N_DEV)
    right = lax.rem(i + 1, N_DEV)

    barrier = pltpu.get_barrier_semaphore()
    for nbr in (left, right):
        pl.semaphore_signal(
            barrier, inc=1, device_id=(nbr,), device_id_type=pl.DeviceIdType.MESH
        )
    pl.semaphore_wait(barrier, 2)

    pl.semaphore_signal(
        credit_sem, inc=2, device_id=(left,), device_id_type=pl.DeviceIdType.MESH
    )

    out_ref[...] = p_ref[...]

    n_steps = 2 * (N_DEV - 1)
    for s in range(n_steps):
        slot = s % 2
        if s < N_DEV - 1:
            send_c = lax.rem(i - s + 2 * N_DEV, N_DEV)
            recv_c = lax.rem(i - s - 1 + 2 * N_DEV, N_DEV)
        else:
            t = s - (N_DEV - 1)
            send_c = lax.rem(i + 1 - t + 2 * N_DEV, N_DEV)
            recv_c = lax.rem(i - t + 2 * N_DEV, N_DEV)
        pl.semaphore_wait(credit_sem, 1)
        rdma = pltpu.make_async_remote_copy(
            src_ref=out_ref.at[send_c],
            dst_ref=comm_ref.at[slot],
            send_sem=send_sems.at[slot],
            recv_sem=recv_sems.at[slot],
            device_id=(right,),
            device_id_type=pl.DeviceIdType.MESH,
        )
        rdma.start()
        rdma.wait()
        if s < N_DEV - 1:
            out_ref[recv_c] = out_ref[recv_c] + comm_ref[slot]
        else:
            out_ref[recv_c] = comm_ref[slot]
        if s < n_steps - 2:
            pl.semaphore_signal(
                credit_sem,
                inc=1,
                device_id=(left,),
                device_id_type=pl.DeviceIdType.MESH,
            )


def kernel(x, Wq, K_ext, V_ext, Wo):
    i = lax.axis_index("i")
    wq_sl = lax.dynamic_slice(Wq, (0, i * DM), (DM, DM))
    wo_sl = lax.dynamic_slice(Wo, (i * DM, 0), (DM, DM))
    xs = x[0]
    k = K_ext[0].transpose(1, 0, 2)
    v = V_ext[0].transpose(1, 0, 2)

    partial = pl.pallas_call(
        _attn_body,
        grid=(HQ_LOC,),
        in_specs=[
            pl.BlockSpec((SQ, DM), lambda h: (0, 0)),
            pl.BlockSpec((DM, DH), lambda h: (0, h)),
            pl.BlockSpec((1, SKV, DH), lambda h: (h, 0, 0)),
            pl.BlockSpec((1, SKV, DH), lambda h: (h, 0, 0)),
            pl.BlockSpec((DH, DM), lambda h: (h, 0)),
        ],
        out_specs=pl.BlockSpec((SQ, DM), lambda h: (0, 0)),
        out_shape=jax.ShapeDtypeStruct((SQ, DM), jnp.float32),
        compiler_params=pltpu.CompilerParams(
            dimension_semantics=("arbitrary",)
        ),
    )(xs, wq_sl, k, v, wo_sl)

    ar_in = partial.reshape(N_DEV, CH, DM)
    out = pl.pallas_call(
        _ar_body,
        out_shape=jax.ShapeDtypeStruct((N_DEV, CH, DM), jnp.float32),
        in_specs=[pl.BlockSpec(memory_space=pltpu.VMEM)],
        out_specs=pl.BlockSpec(memory_space=pltpu.VMEM),
        scratch_shapes=[
            pltpu.VMEM((2, CH, DM), jnp.float32),
            pltpu.SemaphoreType.DMA((2,)),
            pltpu.SemaphoreType.DMA((2,)),
            pltpu.SemaphoreType.REGULAR,
        ],
        compiler_params=pltpu.CompilerParams(collective_id=0),
    )(ar_in)
    return out.reshape(1, SQ, DM)
